# baseline (speedup 1.0000x reference)
import functools

import jax
import jax.numpy as jnp
from jax import lax
from jax.experimental import pallas as pl
from jax.experimental.pallas import tpu as pltpu
from jax.experimental.pallas import tpu_sc as plsc

N = 10000
E = 160000
D = 256
H = D // 2
HA = H + 16
NS = 16
RPT = 632
NPAD = NS * RPT
CH = 64
EPT = 10240
EPAD = EPT * NS
NCH = EPT // CH
BLK = 2000


def _sc_body(feat_lo, feat_hi, src_hbm, dst_hbm, zacc, out_sum,
             acc, src_v, dst_v, rows_a, rows_b, gsem_a, gsem_b):
    c = lax.axis_index("c")
    s = lax.axis_index("s")
    r0 = s * RPT

    pltpu.sync_copy(zacc.at[pl.ds(r0, RPT)], acc.at[pl.ds(r0, RPT)])
    pltpu.sync_copy(src_hbm.at[s], src_v)
    pltpu.sync_copy(dst_hbm.at[s], dst_v)
    plsc.subcore_barrier()

    feat_c = [feat_lo, feat_hi]

    def gather(k, rows, sem):
        @pl.when(c == 0)
        def _():
            pltpu.async_copy(feat_c[0].at[src_v.at[k]], rows, sem)

        @pl.when(c == 1)
        def _():
            pltpu.async_copy(feat_c[1].at[src_v.at[k]], rows, sem)

    def gwait(rows, sem):
        pltpu.make_async_copy(feat_c[0].at[src_v.at[0]], rows, sem).wait()

    def scatter(k, rows):
        pltpu.sync_copy(rows, acc.at[dst_v.at[k]], add=True)

    gather(0, rows_a, gsem_a)
    gather(1, rows_b, gsem_b)

    def pair(i, carry):
        k0 = 2 * i
        gwait(rows_a, gsem_a)
        scatter(k0, rows_a)

        @pl.when(k0 + 2 < NCH)
        def _():
            gather(k0 + 2, rows_a, gsem_a)

        gwait(rows_b, gsem_b)
        scatter(k0 + 1, rows_b)

        @pl.when(k0 + 3 < NCH)
        def _():
            gather(k0 + 3, rows_b, gsem_b)

        return carry

    lax.fori_loop(0, NCH // 2, pair, 0)
    plsc.subcore_barrier()

    pltpu.sync_copy(acc.at[pl.ds(r0, RPT)], out_sum.at[c, pl.ds(r0, RPT)])


_sc_fn = pl.kernel(
    _sc_body,
    out_type=[
        jax.ShapeDtypeStruct((2, NPAD, HA), jnp.float32),
    ],
    mesh=plsc.VectorSubcoreMesh(core_axis_name="c", subcore_axis_name="s"),
    scratch_types=[
        pltpu.VMEM_SHARED((NPAD, HA), jnp.float32),
        pltpu.VMEM((NCH, CH), jnp.int32),
        pltpu.VMEM((NCH, CH), jnp.int32),
        pltpu.VMEM((CH, HA), jnp.float32),
        pltpu.VMEM((CH, HA), jnp.float32),
        pltpu.SemaphoreType.DMA,
        pltpu.SemaphoreType.DMA,
    ],
    compiler_params=pltpu.CompilerParams(use_tc_tiling_on_sc=False),
)


def _tc_body(feat_ref, slo_ref, shi_ref, wst_ref, wnl_ref, wnh_ref, out_ref):
    deg = slo_ref[:, H:H + 1]
    r = 1.0 / jnp.maximum(deg, 1.0)
    acc = jnp.dot(feat_ref[...], wst_ref[...],
                  preferred_element_type=jnp.float32)
    acc = acc + jnp.dot(slo_ref[:, :H] * r, wnl_ref[...],
                        preferred_element_type=jnp.float32)
    acc = acc + jnp.dot(shi_ref[:, :H] * r, wnh_ref[...],
                        preferred_element_type=jnp.float32)
    out_ref[...] = acc


_tc_fn = pl.pallas_call(
    _tc_body,
    grid=(N // BLK,),
    in_specs=[
        pl.BlockSpec((BLK, D), lambda i: (i, 0)),
        pl.BlockSpec((BLK, HA), lambda i: (i, 0)),
        pl.BlockSpec((BLK, HA), lambda i: (i, 0)),
        pl.BlockSpec((D, D), lambda i: (0, 0)),
        pl.BlockSpec((H, D), lambda i: (0, 0)),
        pl.BlockSpec((H, D), lambda i: (0, 0)),
    ],
    out_specs=pl.BlockSpec((BLK, D), lambda i: (i, 0)),
    out_shape=jax.ShapeDtypeStruct((N, D), jnp.float32),
)


def kernel(feat, edge_index, W_self, W_neigh):
    src = edge_index[0].astype(jnp.int32)
    dst = edge_index[1].astype(jnp.int32)
    pad = EPAD - E
    src_p = jnp.concatenate([src, jnp.zeros((pad,), jnp.int32)]).reshape(NS, NCH, CH)
    dst_p = jnp.concatenate([dst, jnp.full((pad,), N + 8, jnp.int32)]).reshape(NS, NCH, CH)
    one = jnp.ones((N, 1), jnp.float32)
    zpad = jnp.zeros((N, HA - H - 1), jnp.float32)
    feat_lo = jnp.concatenate([feat[:, :H], one, zpad], axis=1)
    feat_hi = jnp.concatenate([feat[:, H:], one, zpad], axis=1)
    zacc = jnp.zeros((NPAD, HA), jnp.float32)

    (sums,) = _sc_fn(feat_lo, feat_hi, src_p, dst_p, zacc)

    return _tc_fn(feat, sums[0], sums[1],
                  W_self.T, W_neigh.T[:H], W_neigh.T[H:])


# speedup vs baseline: 1.1907x; 1.1907x over previous
import functools

import jax
import jax.numpy as jnp
from jax import lax
from jax.experimental import pallas as pl
from jax.experimental.pallas import tpu as pltpu
from jax.experimental.pallas import tpu_sc as plsc

N = 10000
E = 160000
D = 256
H = D // 2
NS = 16
RPT = 632
NPAD = NS * RPT
CH = 64
EPT = 10240
EPAD = EPT * NS
NCH = EPT // CH
BLK = 2000


def _sc_body(feat_lo, feat_hi, src_hbm, dst_hbm, zacc, zdeg, ones_hbm,
             out_sum, out_deg,
             acc, dacc, src_v, dst_v, brows_a, brows_b, rows_v, ones_v,
             sem_a, sem_b):
    c = lax.axis_index("c")
    s = lax.axis_index("s")
    r0 = s * RPT

    pltpu.sync_copy(zacc.at[pl.ds(r0, RPT)], acc.at[pl.ds(r0, RPT)])
    pltpu.sync_copy(zdeg.at[pl.ds(r0, RPT)], dacc.at[pl.ds(r0, RPT)])
    pltpu.sync_copy(src_hbm.at[s], src_v)
    pltpu.sync_copy(dst_hbm.at[s], dst_v)
    pltpu.sync_copy(ones_hbm, ones_v)
    plsc.subcore_barrier()

    feat_c = [feat_lo, feat_hi]

    def gather(k, brows, sem):
        @pl.when(c == 0)
        def _():
            pltpu.async_copy(feat_c[0].at[src_v.at[k]], brows, sem)

        @pl.when(c == 1)
        def _():
            pltpu.async_copy(feat_c[1].at[src_v.at[k]], brows, sem)

    def gwait(brows, sem):
        pltpu.make_async_copy(feat_c[0].at[src_v.at[0]], brows, sem).wait()

    def convert(brows):
        hi_mask = jnp.full((16,), -65536, jnp.int32)

        def row(j, carry):
            for g in range(4):
                x = brows[j, pl.ds(g * 16, 16)]
                lo = plsc.bitcast(lax.shift_left(x, 16), jnp.float32)
                hi = plsc.bitcast(lax.bitwise_and(x, hi_mask), jnp.float32)
                rows_v[j, pl.ds(g * 16, 16)] = lo
                rows_v[j, pl.ds(g * 16 + 64, 16)] = hi
            return carry

        lax.fori_loop(0, CH, row, 0)

    def scatter(k):
        pltpu.sync_copy(rows_v, acc.at[dst_v.at[k]], add=True)
        deg_here = jnp.where(c == 0, k < NCH // 2, k >= NCH // 2)

        @pl.when(deg_here)
        def _():
            pltpu.sync_copy(ones_v, dacc.at[dst_v.at[k]], add=True)

    gather(0, brows_a, sem_a)
    gather(1, brows_b, sem_b)

    def pair(i, carry):
        k0 = 2 * i
        gwait(brows_a, sem_a)
        convert(brows_a)
        scatter(k0)

        @pl.when(k0 + 2 < NCH)
        def _():
            gather(k0 + 2, brows_a, sem_a)

        gwait(brows_b, sem_b)
        convert(brows_b)
        scatter(k0 + 1)

        @pl.when(k0 + 3 < NCH)
        def _():
            gather(k0 + 3, brows_b, sem_b)

        return carry

    lax.fori_loop(0, NCH // 2, pair, 0)
    plsc.subcore_barrier()

    pltpu.sync_copy(acc.at[pl.ds(r0, RPT)], out_sum.at[c, pl.ds(r0, RPT)])
    pltpu.sync_copy(dacc.at[pl.ds(r0, RPT)], out_deg.at[c, pl.ds(r0, RPT)])


_sc_fn = pl.kernel(
    _sc_body,
    out_type=[
        jax.ShapeDtypeStruct((2, NPAD, H), jnp.float32),
        jax.ShapeDtypeStruct((2, NPAD, 16), jnp.float32),
    ],
    mesh=plsc.VectorSubcoreMesh(core_axis_name="c", subcore_axis_name="s"),
    scratch_types=[
        pltpu.VMEM_SHARED((NPAD, H), jnp.float32),
        pltpu.VMEM_SHARED((NPAD, 16), jnp.float32),
        pltpu.VMEM((NCH, CH), jnp.int32),
        pltpu.VMEM((NCH, CH), jnp.int32),
        pltpu.VMEM((CH, H // 2), jnp.int32),
        pltpu.VMEM((CH, H // 2), jnp.int32),
        pltpu.VMEM((CH, H), jnp.float32),
        pltpu.VMEM((CH, 16), jnp.float32),
        pltpu.SemaphoreType.DMA,
        pltpu.SemaphoreType.DMA,
    ],
    compiler_params=pltpu.CompilerParams(use_tc_tiling_on_sc=False,
                                         needs_layout_passes=False),
)


def _tc_body(feat_ref, slo_ref, shi_ref, d0_ref, d1_ref,
             wst_ref, wnl_ref, wnh_ref, out_ref):
    deg = d0_ref[:, 0:1] + d1_ref[:, 0:1]
    r = 1.0 / jnp.maximum(deg, 1.0)
    acc = jnp.dot(feat_ref[...], wst_ref[...],
                  preferred_element_type=jnp.float32)
    acc = acc + jnp.dot(slo_ref[...] * r, wnl_ref[...],
                        preferred_element_type=jnp.float32)
    acc = acc + jnp.dot(shi_ref[...] * r, wnh_ref[...],
                        preferred_element_type=jnp.float32)
    out_ref[...] = acc


_tc_fn = pl.pallas_call(
    _tc_body,
    grid=(N // BLK,),
    in_specs=[
        pl.BlockSpec((BLK, D), lambda i: (i, 0)),
        pl.BlockSpec((BLK, H), lambda i: (i, 0)),
        pl.BlockSpec((BLK, H), lambda i: (i, 0)),
        pl.BlockSpec((BLK, 16), lambda i: (i, 0)),
        pl.BlockSpec((BLK, 16), lambda i: (i, 0)),
        pl.BlockSpec((D, D), lambda i: (0, 0)),
        pl.BlockSpec((H, D), lambda i: (0, 0)),
        pl.BlockSpec((H, D), lambda i: (0, 0)),
    ],
    out_specs=pl.BlockSpec((BLK, D), lambda i: (i, 0)),
    out_shape=jax.ShapeDtypeStruct((N, D), jnp.float32),
)


def kernel(feat, edge_index, W_self, W_neigh):
    src = edge_index[0].astype(jnp.int32)
    dst = edge_index[1].astype(jnp.int32)
    pad = EPAD - E
    src_p = jnp.concatenate([src, jnp.zeros((pad,), jnp.int32)]).reshape(NS, NCH, CH)
    dst_p = jnp.concatenate([dst, jnp.full((pad,), N + 8, jnp.int32)]).reshape(NS, NCH, CH)
    fb = lax.bitcast_convert_type(feat.astype(jnp.bfloat16), jnp.uint16)
    fb = fb.astype(jnp.uint32)

    def pack_half(x):
        w = x[:, :H // 2] | (x[:, H // 2:] << 16)
        return lax.bitcast_convert_type(w, jnp.int32)

    feat_lo = pack_half(fb[:, :H])
    feat_hi = pack_half(fb[:, H:])
    zacc = jnp.zeros((NPAD, H), jnp.float32)
    zdeg = jnp.zeros((NPAD, 16), jnp.float32)
    ones = jnp.ones((CH, 16), jnp.float32)

    sums, degs = _sc_fn(feat_lo, feat_hi, src_p, dst_p, zacc, zdeg, ones)

    return _tc_fn(feat, sums[0], sums[1], degs[0], degs[1],
                  W_self.T, W_neigh.T[:H], W_neigh.T[H:])
